# baseline (speedup 1.0000x reference)
import functools

import jax
import jax.numpy as jnp
from jax import lax
from jax.experimental import pallas as pl
from jax.experimental.pallas import tpu as pltpu
from jax.experimental.pallas import tpu_sc as plsc

VOCAB = 100000
EMBED = 128
BATCH = 1024
HIST = 200
TOTAL = BATCH * HIST

NC = 2
NS = 16
NW = NC * NS
B_PER_W = TOTAL // NW
CHUNK = 128
N_GROUPS = B_PER_W // CHUNK
DEPTH = 6
LOOKAHEAD = 3

_mesh = plsc.VectorSubcoreMesh(core_axis_name="c", subcore_axis_name="s")


@functools.partial(
    pl.kernel,
    mesh=_mesh,
    out_type=jax.ShapeDtypeStruct((TOTAL, EMBED), jnp.float32),
    scratch_types=(
        [pltpu.VMEM((N_GROUPS, CHUNK), jnp.int32)]
        + [pltpu.VMEM((CHUNK, EMBED), jnp.float32)] * DEPTH
        + [pltpu.SemaphoreType.DMA] * (2 * DEPTH)
    ),
)
def _gather_kernel(idx_hbm, table_hbm, out_hbm, idx_v, *bufs_and_sems):
    bufs = bufs_and_sems[:DEPTH]
    sg = bufs_and_sems[DEPTH:2 * DEPTH]
    so = bufs_and_sems[2 * DEPTH:]

    wid = lax.axis_index("s") * NC + lax.axis_index("c")
    base = wid * B_PER_W

    pltpu.sync_copy(idx_hbm.at[wid], idx_v)

    def fire_gather(g, r):
        pltpu.async_copy(table_hbm.at[idx_v.at[g]], bufs[r], sg[r])

    def out_slc(g):
        return out_hbm.at[pl.ds(base + g * CHUNK, CHUNK)]

    def step(g, r):
        ra = (r + LOOKAHEAD) % DEPTH

        @pl.when(g >= DEPTH - LOOKAHEAD)
        def _():
            pltpu.make_async_copy(bufs[ra], out_slc(0), so[ra]).wait()

        @pl.when(g < N_GROUPS - LOOKAHEAD)
        def _():
            fire_gather(g + LOOKAHEAD, ra)

        pltpu.make_async_copy(table_hbm.at[pl.ds(0, CHUNK)],
                              bufs[r], sg[r]).wait()
        pltpu.async_copy(bufs[r], out_slc(g), so[r])

    for g in range(LOOKAHEAD):
        fire_gather(g, g)

    def body(g, carry):
        for r in range(DEPTH):
            @pl.when(g % DEPTH == r)
            def _(r=r):
                step(g, r)
        return carry

    lax.fori_loop(0, N_GROUPS, body, 0)

    for g in range(N_GROUPS - (DEPTH - LOOKAHEAD), N_GROUPS):
        r = g % DEPTH
        pltpu.make_async_copy(bufs[r], out_slc(0), so[r]).wait()


def kernel(input, table):
    idx = input.reshape(TOTAL).astype(jnp.int32)
    out = _gather_kernel(idx.reshape(NW, N_GROUPS, CHUNK), table)
    return out.reshape(BATCH, HIST, EMBED)


# speedup vs baseline: 1.0023x; 1.0023x over previous
import functools

import jax
import jax.numpy as jnp
from jax import lax
from jax.experimental import pallas as pl
from jax.experimental.pallas import tpu as pltpu
from jax.experimental.pallas import tpu_sc as plsc

VOCAB = 100000
EMBED = 128
BATCH = 1024
HIST = 200
TOTAL = BATCH * HIST

NC = 2
NS = 16
NW = NC * NS
B_PER_W = TOTAL // NW
CHUNK = 128
N_GROUPS = B_PER_W // CHUNK
DEPTH = 6
LOOKAHEAD = 4

_mesh = plsc.VectorSubcoreMesh(core_axis_name="c", subcore_axis_name="s")


@functools.partial(
    pl.kernel,
    mesh=_mesh,
    out_type=jax.ShapeDtypeStruct((TOTAL, EMBED), jnp.float32),
    scratch_types=(
        [pltpu.VMEM((N_GROUPS, CHUNK), jnp.int32)]
        + [pltpu.VMEM((CHUNK, EMBED), jnp.float32)] * DEPTH
        + [pltpu.SemaphoreType.DMA] * (2 * DEPTH)
    ),
)
def _gather_kernel(idx_hbm, table_hbm, out_hbm, idx_v, *bufs_and_sems):
    bufs = bufs_and_sems[:DEPTH]
    sg = bufs_and_sems[DEPTH:2 * DEPTH]
    so = bufs_and_sems[2 * DEPTH:]

    wid = lax.axis_index("s") * NC + lax.axis_index("c")
    base = wid * B_PER_W

    pltpu.sync_copy(idx_hbm.at[wid], idx_v)

    def fire_gather(g, r):
        pltpu.async_copy(table_hbm.at[idx_v.at[g]], bufs[r], sg[r])

    def out_slc(g):
        return out_hbm.at[pl.ds(base + g * CHUNK, CHUNK)]

    def step(g, r):
        ra = (r + LOOKAHEAD) % DEPTH

        @pl.when(g >= DEPTH - LOOKAHEAD)
        def _():
            pltpu.make_async_copy(bufs[ra], out_slc(0), so[ra]).wait()

        @pl.when(g < N_GROUPS - LOOKAHEAD)
        def _():
            fire_gather(g + LOOKAHEAD, ra)

        pltpu.make_async_copy(table_hbm.at[pl.ds(0, CHUNK)],
                              bufs[r], sg[r]).wait()
        pltpu.async_copy(bufs[r], out_slc(g), so[r])

    for g in range(LOOKAHEAD):
        fire_gather(g, g)

    def body(g, carry):
        for r in range(DEPTH):
            @pl.when(g % DEPTH == r)
            def _(r=r):
                step(g, r)
        return carry

    lax.fori_loop(0, N_GROUPS, body, 0)

    for g in range(N_GROUPS - (DEPTH - LOOKAHEAD), N_GROUPS):
        r = g % DEPTH
        pltpu.make_async_copy(bufs[r], out_slc(0), so[r]).wait()


def kernel(input, table):
    idx = input.reshape(TOTAL).astype(jnp.int32)
    out = _gather_kernel(idx.reshape(NW, N_GROUPS, CHUNK), table)
    return out.reshape(BATCH, HIST, EMBED)


# speedup vs baseline: 1.0091x; 1.0069x over previous
import functools

import jax
import jax.numpy as jnp
from jax import lax
from jax.experimental import pallas as pl
from jax.experimental.pallas import tpu as pltpu
from jax.experimental.pallas import tpu_sc as plsc

VOCAB = 100000
EMBED = 128
BATCH = 1024
HIST = 200
TOTAL = BATCH * HIST

NC = 2
NS = 16
NW = NC * NS
B_PER_W = TOTAL // NW
CHUNK = 128
N_GROUPS = B_PER_W // CHUNK
DEPTH = 7
LOOKAHEAD = 5

_mesh = plsc.VectorSubcoreMesh(core_axis_name="c", subcore_axis_name="s")


@functools.partial(
    pl.kernel,
    mesh=_mesh,
    out_type=jax.ShapeDtypeStruct((TOTAL, EMBED), jnp.float32),
    scratch_types=(
        [pltpu.VMEM((N_GROUPS, CHUNK), jnp.int32)]
        + [pltpu.VMEM((CHUNK, EMBED), jnp.float32)] * DEPTH
        + [pltpu.SemaphoreType.DMA] * (2 * DEPTH)
    ),
)
def _gather_kernel(idx_hbm, table_hbm, out_hbm, idx_v, *bufs_and_sems):
    bufs = bufs_and_sems[:DEPTH]
    sg = bufs_and_sems[DEPTH:2 * DEPTH]
    so = bufs_and_sems[2 * DEPTH:]

    wid = lax.axis_index("s") * NC + lax.axis_index("c")
    base = wid * B_PER_W

    pltpu.sync_copy(idx_hbm.at[wid], idx_v)

    def fire_gather(g, r):
        pltpu.async_copy(table_hbm.at[idx_v.at[g]], bufs[r], sg[r])

    def out_slc(g):
        return out_hbm.at[pl.ds(base + g * CHUNK, CHUNK)]

    def step(g, r):
        ra = (r + LOOKAHEAD) % DEPTH

        @pl.when(g >= DEPTH - LOOKAHEAD)
        def _():
            pltpu.make_async_copy(bufs[ra], out_slc(0), so[ra]).wait()

        @pl.when(g < N_GROUPS - LOOKAHEAD)
        def _():
            fire_gather(g + LOOKAHEAD, ra)

        pltpu.make_async_copy(table_hbm.at[pl.ds(0, CHUNK)],
                              bufs[r], sg[r]).wait()
        pltpu.async_copy(bufs[r], out_slc(g), so[r])

    for g in range(LOOKAHEAD):
        fire_gather(g, g)

    def body(g, carry):
        for r in range(DEPTH):
            @pl.when(g % DEPTH == r)
            def _(r=r):
                step(g, r)
        return carry

    lax.fori_loop(0, N_GROUPS, body, 0)

    for g in range(N_GROUPS - (DEPTH - LOOKAHEAD), N_GROUPS):
        r = g % DEPTH
        pltpu.make_async_copy(bufs[r], out_slc(0), so[r]).wait()


def kernel(input, table):
    idx = input.reshape(TOTAL).astype(jnp.int32)
    out = _gather_kernel(idx.reshape(NW, N_GROUPS, CHUNK), table)
    return out.reshape(BATCH, HIST, EMBED)
